# baseline (speedup 1.0000x reference)
import functools

import jax
import jax.numpy as jnp
from jax import lax
from jax.experimental import pallas as pl
from jax.experimental.pallas import tpu as pltpu
from jax.experimental.pallas import tpu_sc as plsc

_NC = 2
_NS = 16
_NW = _NC * _NS
_LANES = 16
_NBUF = 2
_BSLAB = 128


@functools.lru_cache(maxsize=None)
def _build_pad(v, d, bk):
    def body(t_ref, o_ref):
        o_ref[:, 0:d] = t_ref[...]

    return pl.pallas_call(
        body,
        grid=(v // bk,),
        in_specs=[pl.BlockSpec((bk, d), lambda i: (i, 0))],
        out_specs=pl.BlockSpec((bk, 128), lambda i: (i, 0)),
        out_shape=jax.ShapeDtypeStruct((v, 128), jnp.float32),
    )


@functools.lru_cache(maxsize=None)
def _build(nb, s, d):
    n_slab = nb // _NW
    assert n_slab == _BSLAB and s % _NBUF == 0 and s // _NBUF >= 2
    assert d % _LANES == 0
    mesh = plsc.VectorSubcoreMesh(
        core_axis_name="c", subcore_axis_name="s",
        num_cores=_NC, num_subcores=_NS,
    )

    scratch = (
        tuple(pltpu.VMEM((_BSLAB,), jnp.int32) for _ in range(_NBUF)),
        tuple(pltpu.VMEM((_BSLAB, 128), jnp.float32) for _ in range(_NBUF)),
        tuple(pltpu.VMEM((d, _BSLAB), jnp.float32) for _ in range(_NBUF)),
        pltpu.VMEM((s, d), jnp.float32),
        tuple(pltpu.SemaphoreType.DMA for _ in range(_NBUF)),
        tuple(pltpu.SemaphoreType.DMA for _ in range(_NBUF)),
        tuple(pltpu.SemaphoreType.DMA for _ in range(_NBUF)),
    )

    @functools.partial(
        pl.kernel,
        out_type=jax.ShapeDtypeStruct((s, d, nb), jnp.float32),
        mesh=mesh,
        scratch_types=scratch,
        compiler_params=pltpu.CompilerParams(needs_layout_passes=False),
    )
    def emb_kernel(tok_hbm, table_hbm, pos_hbm, out_hbm,
                   idx_v, rows_v, out_v, pos_v, gsem, ssem, isem):
        wid = lax.axis_index("s") * _NC + lax.axis_index("c")
        b0 = wid * _BSLAB
        pltpu.sync_copy(pos_hbm, pos_v)
        lane = jax.lax.iota(jnp.int32, 16)

        def start_idx(g, b):
            pltpu.async_copy(tok_hbm.at[g, pl.ds(b0, _BSLAB)], idx_v[b], isem[b])

        def wait_idx(b):
            pltpu.make_async_copy(tok_hbm.at[0, pl.ds(0, _BSLAB)], idx_v[b], isem[b]).wait()

        def start_gather(b):
            pltpu.async_copy(table_hbm.at[idx_v[b]], rows_v[b], gsem[b])

        def wait_gather(b):
            pltpu.make_async_copy(table_hbm.at[idx_v[b]], rows_v[b], gsem[b]).wait()

        def start_store(g, b):
            pltpu.async_copy(out_v[b], out_hbm.at[g, :, pl.ds(b0, _BSLAB)], ssem[b])

        def wait_store(b):
            pltpu.make_async_copy(out_v[b], out_hbm.at[0, :, pl.ds(0, _BSLAB)], ssem[b]).wait()

        def compute(g, b):
            rows_b, out_b = rows_v[b], out_v[b]
            gvec = jnp.full((16,), g, jnp.int32)

            def dim_body(dd, carry):
                dvec = jnp.full((16,), dd, jnp.int32)
                p = plsc.load_gather(pos_v, [gvec, dvec])
                for j in range(_BSLAB // _LANES):
                    tvec = lane + (j * _LANES)
                    vals = plsc.load_gather(rows_b, [tvec, dvec])
                    out_b[dd, pl.ds(j * _LANES, _LANES)] = vals + p
                return carry

            lax.fori_loop(0, d, dim_body, 0)

        def step(g, b, *, idx_next=True, store_wait=True, gather_next=True):
            wait_gather(b)
            if idx_next:
                start_idx(g + _NBUF, b)
            if store_wait:
                wait_store((b + _NBUF - 1) % _NBUF)
            if gather_next:
                hb = (b + _NBUF - 1) % _NBUF
                wait_idx(hb)
                start_gather(hb)
            compute(g, b)
            start_store(g, b)

        for b in range(_NBUF):
            start_idx(b, b)
        for b in range(_NBUF - 1):
            wait_idx(b)
            start_gather(b)
        step(0, 0, store_wait=False)
        for b in range(1, _NBUF):
            step(b, b)

        n_super = s // _NBUF

        def super_body(go, carry):
            g0 = go * _NBUF
            for b in range(_NBUF):
                step(g0 + b, b)
            return carry

        lax.fori_loop(1, n_super - 1, super_body, 0)

        g0 = (n_super - 1) * _NBUF
        step(g0, 0, idx_next=False)
        for b in range(1, _NBUF):
            step(g0 + b, b, idx_next=False, gather_next=False)
        wait_store(_NBUF - 1)

    return emb_kernel


def kernel(tokens, token_embedding, position_embedding):
    nb, s = tokens.shape
    v, d = token_embedding.shape
    table128 = _build_pad(v, d, 4000)(token_embedding)
    fn = _build(nb, s, d)
    out_t = fn(tokens.T.astype(jnp.int32), table128, position_embedding[:s])
    return out_t.transpose(2, 0, 1)


# speedup vs baseline: 1.4430x; 1.4430x over previous
import functools

import jax
import jax.numpy as jnp
from jax import lax
from jax.experimental import pallas as pl
from jax.experimental.pallas import tpu as pltpu
from jax.experimental.pallas import tpu_sc as plsc

_NC = 2
_NS = 16
_NW = _NC * _NS
_LANES = 16
_NBUF = 2
_BSLAB = 128


@functools.lru_cache(maxsize=None)
def _build_pad(v, d, bk):
    def body(t_ref, o_ref):
        o_ref[:, 0:d] = t_ref[...]

    return pl.pallas_call(
        body,
        grid=(v // bk,),
        in_specs=[pl.BlockSpec((bk, d), lambda i: (i, 0))],
        out_specs=pl.BlockSpec((bk, 128), lambda i: (i, 0)),
        out_shape=jax.ShapeDtypeStruct((v, 128), jnp.float32),
    )


@functools.lru_cache(maxsize=None)
def _build(nb, s, d):
    n_slab = nb // _NW
    assert n_slab == _BSLAB and s % _NBUF == 0 and s // _NBUF >= 2
    assert d % _LANES == 0
    mesh = plsc.VectorSubcoreMesh(
        core_axis_name="c", subcore_axis_name="s",
        num_cores=_NC, num_subcores=_NS,
    )

    pitch = d + 1
    scratch = (
        tuple(pltpu.VMEM((_BSLAB,), jnp.int32) for _ in range(_NBUF)),
        tuple(pltpu.VMEM((_BSLAB, 128), jnp.float32) for _ in range(_NBUF)),
        tuple(pltpu.VMEM((_BSLAB * pitch,), jnp.float32) for _ in range(_NBUF)),
        tuple(pltpu.VMEM((d, _BSLAB), jnp.float32) for _ in range(_NBUF)),
        pltpu.VMEM((s, d), jnp.float32),
        tuple(pltpu.SemaphoreType.DMA for _ in range(_NBUF)),
        tuple(pltpu.SemaphoreType.DMA for _ in range(_NBUF)),
        tuple(pltpu.SemaphoreType.DMA for _ in range(_NBUF)),
    )

    @functools.partial(
        pl.kernel,
        out_type=jax.ShapeDtypeStruct((s, d, nb), jnp.float32),
        mesh=mesh,
        scratch_types=scratch,
        compiler_params=pltpu.CompilerParams(needs_layout_passes=False),
    )
    def emb_kernel(tok_hbm, table_hbm, pos_hbm, out_hbm,
                   idx_v, rows_v, pad_v, out_v, pos_v, gsem, ssem, isem):
        wid = lax.axis_index("s") * _NC + lax.axis_index("c")
        b0 = wid * _BSLAB
        pltpu.sync_copy(pos_hbm, pos_v)
        lane = jax.lax.iota(jnp.int32, 16)

        def start_idx(g, b):
            pltpu.async_copy(tok_hbm.at[g, pl.ds(b0, _BSLAB)], idx_v[b], isem[b])

        def wait_idx(b):
            pltpu.make_async_copy(tok_hbm.at[0, pl.ds(0, _BSLAB)], idx_v[b], isem[b]).wait()

        def start_gather(b):
            pltpu.async_copy(table_hbm.at[idx_v[b]], rows_v[b], gsem[b])

        def wait_gather(b):
            pltpu.make_async_copy(table_hbm.at[idx_v[b]], rows_v[b], gsem[b]).wait()

        def start_store(g, b):
            pltpu.async_copy(out_v[b], out_hbm.at[g, :, pl.ds(b0, _BSLAB)], ssem[b])

        def wait_store(b):
            pltpu.make_async_copy(out_v[b], out_hbm.at[0, :, pl.ds(0, _BSLAB)], ssem[b]).wait()

        def compute(g, b):
            rows_b, pad_b, out_b = rows_v[b], pad_v[b], out_v[b]
            pvecs = [pos_v[g, pl.ds(c * _LANES, _LANES)] for c in range(d // _LANES)]

            def row_body(r, carry):
                rbase = r * pitch
                for c in range(d // _LANES):
                    vals = rows_b[r, pl.ds(c * _LANES, _LANES)] + pvecs[c]
                    plsc.store_scatter(pad_b, [rbase + c * _LANES + lane], vals)
                return carry

            lax.fori_loop(0, _BSLAB, row_body, 0, unroll=2)

            t129 = [(lane + j * _LANES) * pitch for j in range(_BSLAB // _LANES)]

            def dim_body(dd, carry):
                for j in range(_BSLAB // _LANES):
                    vals = plsc.load_gather(pad_b, [t129[j] + dd])
                    out_b[dd, pl.ds(j * _LANES, _LANES)] = vals
                return carry

            lax.fori_loop(0, d, dim_body, 0, unroll=2)

        def step(g, b, *, idx_next=True, store_wait=True, gather_next=True):
            wait_gather(b)
            if idx_next:
                start_idx(g + _NBUF, b)
            if store_wait:
                wait_store((b + _NBUF - 1) % _NBUF)
            if gather_next:
                hb = (b + _NBUF - 1) % _NBUF
                wait_idx(hb)
                start_gather(hb)
            compute(g, b)
            start_store(g, b)

        for b in range(_NBUF):
            start_idx(b, b)
        for b in range(_NBUF - 1):
            wait_idx(b)
            start_gather(b)
        step(0, 0, store_wait=False)
        for b in range(1, _NBUF):
            step(b, b)

        n_super = s // _NBUF

        def super_body(go, carry):
            g0 = go * _NBUF
            for b in range(_NBUF):
                step(g0 + b, b)
            return carry

        lax.fori_loop(1, n_super - 1, super_body, 0)

        g0 = (n_super - 1) * _NBUF
        step(g0, 0, idx_next=False)
        for b in range(1, _NBUF):
            step(g0 + b, b, idx_next=False, gather_next=False)
        wait_store(_NBUF - 1)

    return emb_kernel


def kernel(tokens, token_embedding, position_embedding):
    nb, s = tokens.shape
    v, d = token_embedding.shape
    table128 = _build_pad(v, d, 4000)(token_embedding)
    fn = _build(nb, s, d)
    out_t = fn(tokens.T.astype(jnp.int32), table128, position_embedding[:s])
    return out_t.transpose(2, 0, 1)


# speedup vs baseline: 1.9975x; 1.3843x over previous
import functools

import jax
import jax.numpy as jnp
from jax import lax
from jax.experimental import pallas as pl
from jax.experimental.pallas import tpu as pltpu
from jax.experimental.pallas import tpu_sc as plsc

_NC = 2
_NS = 16
_NW = _NC * _NS
_LANES = 16
_NBUF = 4


@functools.lru_cache(maxsize=None)
def _build(n_seq, s, d):
    ch = s
    n_per_w = n_seq // _NW
    n_super = n_per_w // _NBUF
    assert n_seq % _NW == 0 and n_per_w % _NBUF == 0 and n_super >= 2
    assert d % _LANES == 0
    mesh = plsc.VectorSubcoreMesh(
        core_axis_name="c", subcore_axis_name="s",
        num_cores=_NC, num_subcores=_NS,
    )

    scratch = (
        tuple(pltpu.VMEM((ch,), jnp.int32) for _ in range(_NBUF)),
        tuple(pltpu.VMEM((ch, d), jnp.float32) for _ in range(_NBUF)),
        pltpu.VMEM((s, d), jnp.float32),
        tuple(pltpu.SemaphoreType.DMA for _ in range(_NBUF)),
        tuple(pltpu.SemaphoreType.DMA for _ in range(_NBUF)),
        tuple(pltpu.SemaphoreType.DMA for _ in range(_NBUF)),
    )

    @functools.partial(
        pl.kernel,
        out_type=jax.ShapeDtypeStruct((n_seq * s, d), jnp.float32),
        mesh=mesh,
        scratch_types=scratch,
        compiler_params=pltpu.CompilerParams(use_tc_tiling_on_sc=False),
    )
    def emb_kernel(tok_hbm, table_hbm, pos_hbm, out_hbm,
                   idx_v, rows_v, pos_v, gsem, ssem, isem):
        wid = lax.axis_index("s") * _NC + lax.axis_index("c")
        seq0 = wid * n_per_w
        base = seq0 * ch
        pltpu.sync_copy(pos_hbm, pos_v)

        def start_idx(g, b):
            pltpu.async_copy(tok_hbm.at[seq0 + g], idx_v[b], isem[b])

        def wait_idx(b):
            pltpu.make_async_copy(tok_hbm.at[0], idx_v[b], isem[b]).wait()

        def start_gather(b):
            pltpu.async_copy(table_hbm.at[idx_v[b]], rows_v[b], gsem[b])

        def wait_gather(b):
            pltpu.make_async_copy(table_hbm.at[idx_v[b]], rows_v[b], gsem[b]).wait()

        def start_store(g, b):
            pltpu.async_copy(rows_v[b], out_hbm.at[pl.ds(base + g * ch, ch)], ssem[b])

        def wait_store(b):
            pltpu.make_async_copy(rows_v[b], out_hbm.at[pl.ds(0, ch)], ssem[b]).wait()

        def compute(b):
            rows_b = rows_v[b]

            def row_body(r, carry):
                for c in range(d // _LANES):
                    sl = pl.ds(c * _LANES, _LANES)
                    rows_b[r, sl] = rows_b[r, sl] + pos_v[r, sl]
                return carry

            lax.fori_loop(0, ch, row_body, 0)

        def step(g, b, *, idx_next=True, store_wait=True, gather_next=True):
            wait_gather(b)
            if idx_next:
                start_idx(g + _NBUF, b)
            if store_wait:
                wait_store((b + _NBUF - 1) % _NBUF)
            if gather_next:
                hb = (b + _NBUF - 1) % _NBUF
                wait_idx(hb)
                start_gather(hb)
            compute(b)
            start_store(g, b)

        for b in range(_NBUF):
            start_idx(b, b)
        for b in range(_NBUF - 1):
            wait_idx(b)
            start_gather(b)
        step(0, 0, store_wait=False)
        for b in range(1, _NBUF):
            step(b, b)

        def super_body(go, carry):
            g0 = go * _NBUF
            for b in range(_NBUF):
                step(g0 + b, b)
            return carry

        lax.fori_loop(1, n_super - 1, super_body, 0)

        g0 = (n_super - 1) * _NBUF
        step(g0, 0, idx_next=False)
        for b in range(1, _NBUF):
            step(g0 + b, b, idx_next=False, gather_next=False)
        wait_store(_NBUF - 1)

    return emb_kernel


def kernel(tokens, token_embedding, position_embedding):
    nb, s = tokens.shape
    v, d = token_embedding.shape
    fn = _build(nb, s, d)
    out = fn(tokens.astype(jnp.int32), token_embedding, position_embedding[:s])
    return out.reshape(nb, s, d)


# speedup vs baseline: 2.0006x; 1.0015x over previous
import functools

import jax
import jax.numpy as jnp
from jax import lax
from jax.experimental import pallas as pl
from jax.experimental.pallas import tpu as pltpu
from jax.experimental.pallas import tpu_sc as plsc

_NC = 2
_NS = 16
_NW = _NC * _NS
_LANES = 16
_NBUF = 4


@functools.lru_cache(maxsize=None)
def _build(n_seq, s, d):
    ch = s
    n_per_w = n_seq // _NW
    n_super = n_per_w // _NBUF
    assert n_seq % _NW == 0 and n_per_w % _NBUF == 0 and n_super >= 2
    assert d % _LANES == 0
    mesh = plsc.VectorSubcoreMesh(
        core_axis_name="c", subcore_axis_name="s",
        num_cores=_NC, num_subcores=_NS,
    )

    scratch = (
        tuple(pltpu.VMEM((ch,), jnp.int32) for _ in range(_NBUF)),
        tuple(pltpu.VMEM((ch, d), jnp.float32) for _ in range(_NBUF)),
        pltpu.VMEM((s, d), jnp.float32),
        tuple(pltpu.SemaphoreType.DMA for _ in range(_NBUF)),
        tuple(pltpu.SemaphoreType.DMA for _ in range(_NBUF)),
        tuple(pltpu.SemaphoreType.DMA for _ in range(_NBUF)),
    )

    @functools.partial(
        pl.kernel,
        out_type=jax.ShapeDtypeStruct((n_seq, s, d), jnp.float32),
        mesh=mesh,
        scratch_types=scratch,
        compiler_params=pltpu.CompilerParams(use_tc_tiling_on_sc=False),
    )
    def emb_kernel(tok_hbm, table_hbm, pos_hbm, out_hbm,
                   idx_v, rows_v, pos_v, gsem, ssem, isem):
        wid = lax.axis_index("s") * _NC + lax.axis_index("c")
        seq0 = wid * n_per_w
        pltpu.sync_copy(pos_hbm, pos_v)

        def start_idx(g, b):
            pltpu.async_copy(tok_hbm.at[seq0 + g], idx_v[b], isem[b])

        def wait_idx(b):
            pltpu.make_async_copy(tok_hbm.at[0], idx_v[b], isem[b]).wait()

        def start_gather(b):
            pltpu.async_copy(table_hbm.at[idx_v[b]], rows_v[b], gsem[b])

        def wait_gather(b):
            pltpu.make_async_copy(table_hbm.at[idx_v[b]], rows_v[b], gsem[b]).wait()

        def start_store(g, b):
            pltpu.async_copy(rows_v[b], out_hbm.at[seq0 + g], ssem[b])

        def wait_store(b):
            pltpu.make_async_copy(rows_v[b], out_hbm.at[0], ssem[b]).wait()

        def compute(b):
            rows_b = rows_v[b]

            def row_body(r, carry):
                for c in range(d // _LANES):
                    sl = pl.ds(c * _LANES, _LANES)
                    rows_b[r, sl] = rows_b[r, sl] + pos_v[r, sl]
                return carry

            lax.fori_loop(0, ch, row_body, 0)

        def step(g, b, *, idx_next=True, store_wait=True, gather_next=True):
            wait_gather(b)
            if idx_next:
                start_idx(g + _NBUF, b)
            if store_wait:
                wait_store((b + _NBUF - 1) % _NBUF)
            if gather_next:
                hb = (b + _NBUF - 1) % _NBUF
                wait_idx(hb)
                start_gather(hb)
            compute(b)
            start_store(g, b)

        for b in range(_NBUF):
            start_idx(b, b)
        for b in range(_NBUF - 1):
            wait_idx(b)
            start_gather(b)
        step(0, 0, store_wait=False)
        for b in range(1, _NBUF):
            step(b, b)

        def super_body(go, carry):
            g0 = go * _NBUF
            for b in range(_NBUF):
                step(g0 + b, b)
            return carry

        lax.fori_loop(1, n_super - 1, super_body, 0)

        g0 = (n_super - 1) * _NBUF
        step(g0, 0, idx_next=False)
        for b in range(1, _NBUF):
            step(g0 + b, b, idx_next=False, gather_next=False)
        wait_store(_NBUF - 1)

    return emb_kernel


def kernel(tokens, token_embedding, position_embedding):
    nb, s = tokens.shape
    v, d = token_embedding.shape
    fn = _build(nb, s, d)
    return fn(tokens.astype(jnp.int32), token_embedding, position_embedding[:s])


# speedup vs baseline: 2.7146x; 1.3569x over previous
import functools

import jax
import jax.numpy as jnp
from jax import lax
from jax.experimental import pallas as pl
from jax.experimental.pallas import tpu as pltpu
from jax.experimental.pallas import tpu_sc as plsc

_NC = 2
_NS = 16
_NW = _NC * _NS
_LANES = 16
_NBUF = 2
_CH = 200


@functools.lru_cache(maxsize=None)
def _build(n_rows, d, s):
    ch = _CH
    n_per_w = n_rows // _NW
    n_chunks = n_per_w // ch
    n_super = n_chunks // _NBUF
    assert n_per_w % ch == 0 and n_chunks % _NBUF == 0 and n_super >= 2
    assert ch == s
    mesh = plsc.VectorSubcoreMesh(
        core_axis_name="c", subcore_axis_name="s",
        num_cores=_NC, num_subcores=_NS,
    )

    scratch = (
        tuple(pltpu.VMEM((ch,), jnp.int32) for _ in range(_NBUF)),
        tuple(pltpu.VMEM((ch, 128), jnp.float32) for _ in range(_NBUF)),
        tuple(pltpu.VMEM((ch, d), jnp.float32) for _ in range(_NBUF)),
        pltpu.VMEM((s, d), jnp.float32),
        tuple(pltpu.SemaphoreType.DMA for _ in range(_NBUF)),
        tuple(pltpu.SemaphoreType.DMA for _ in range(_NBUF)),
        tuple(pltpu.SemaphoreType.DMA for _ in range(_NBUF)),
    )

    @functools.partial(
        pl.kernel,
        out_type=jax.ShapeDtypeStruct((n_rows, d), jnp.float32),
        mesh=mesh,
        scratch_types=scratch,
    )
    def emb_kernel(tok_hbm, table_hbm, pos_hbm, out_hbm,
                   idx_v, rows_v, out_v, pos_v, gsem, ssem, isem):
        wid = lax.axis_index("s") * _NC + lax.axis_index("c")
        base = wid * n_per_w
        pltpu.sync_copy(pos_hbm, pos_v)

        def start_idx(g, b):
            pltpu.async_copy(tok_hbm.at[pl.ds(base + g * ch, ch)], idx_v[b], isem[b])

        def wait_idx(b):
            pltpu.make_async_copy(tok_hbm.at[pl.ds(0, ch)], idx_v[b], isem[b]).wait()

        def start_gather(b):
            pltpu.async_copy(table_hbm.at[idx_v[b]], rows_v[b], gsem[b])

        def wait_gather(b):
            pltpu.make_async_copy(table_hbm.at[idx_v[b]], rows_v[b], gsem[b]).wait()

        def start_store(g, b):
            pltpu.async_copy(out_v[b], out_hbm.at[pl.ds(base + g * ch, ch)], ssem[b])

        def wait_store(b):
            pltpu.make_async_copy(out_v[b], out_hbm.at[pl.ds(0, ch)], ssem[b]).wait()

        def compute(b):
            rows_b, out_b = rows_v[b], out_v[b]

            def row_body(r, carry):
                for c in range(d // _LANES):
                    sl = pl.ds(c * _LANES, _LANES)
                    out_b[r, sl] = rows_b[r, sl] + pos_v[r, sl]
                return carry

            lax.fori_loop(0, ch, row_body, 0)

        def step(g, b, *, idx_next=True, store_wait=True, gather_next=True):
            wait_gather(b)
            if idx_next:
                start_idx(g + _NBUF, b)
            if store_wait:
                wait_store((b + _NBUF - 1) % _NBUF)
            if gather_next:
                hb = (b + _NBUF - 1) % _NBUF
                wait_idx(hb)
                start_gather(hb)
            compute(b)
            start_store(g, b)

        for b in range(_NBUF):
            start_idx(b, b)
        for b in range(_NBUF - 1):
            wait_idx(b)
            start_gather(b)
        step(0, 0, store_wait=False)
        for b in range(1, _NBUF):
            step(b, b)

        def super_body(go, carry):
            g0 = go * _NBUF
            for b in range(_NBUF):
                step(g0 + b, b)
            return carry

        lax.fori_loop(1, n_super - 1, super_body, 0)

        g0 = (n_super - 1) * _NBUF
        step(g0, 0, idx_next=False)
        for b in range(1, _NBUF):
            step(g0 + b, b, idx_next=False, gather_next=False)
        wait_store(_NBUF - 1)

    return emb_kernel


def kernel(tokens, token_embedding, position_embedding):
    b, s = tokens.shape
    _, d = token_embedding.shape
    flat = tokens.reshape(-1).astype(jnp.int32)
    table128 = jnp.pad(token_embedding, ((0, 0), (0, 128 - d)))
    fn = _build(b * s, d, s)
    out = fn(flat, table128, position_embedding[:s])
    return out.reshape(b, s, d)
